# baseline (speedup 1.0000x reference)
import functools

import jax
import jax.numpy as jnp
from jax import lax
from jax.experimental import pallas as pl
from jax.experimental.pallas import tpu as pltpu
from jax.experimental.pallas import tpu_sc as plsc

N = 10000
E = 320000
D = 128

NC = 2
NS = 16
NW = NC * NS
EPW = E // NW
K = 100
CH = EPW // K
BLK = 25
NBLK = CH // BLK
ZR = 25
RPT = N // NS

@functools.lru_cache(maxsize=None)
def _mesh():
    return plsc.VectorSubcoreMesh(
        core_axis_name="c", subcore_axis_name="s",
        num_cores=NC, num_subcores=NS)


def _sc_counts_body(dst3_hbm, out_hbm, idx_v, cnt_v):
    c = lax.axis_index("c")
    s = lax.axis_index("s")
    wid = s * NC + c
    for set_i in range(3):
        pltpu.sync_copy(dst3_hbm.at[set_i, wid], idx_v)

        def _zero(i, _):
            cnt_v[pl.ds(i * 16, 16)] = jnp.zeros((16,), jnp.float32)
            return 0
        lax.fori_loop(0, N // 16, _zero, 0)

        def _count(i, _):
            idx = idx_v[i]
            plsc.addupdate_scatter(cnt_v, [idx], jnp.ones((16,), jnp.float32))
            return 0
        lax.fori_loop(0, EPW // 16, _count, 0)
        pltpu.sync_copy(cnt_v, out_hbm.at[set_i, wid])


@jax.jit
def _sc_counts(dst3):
    kern = pl.kernel(
        _sc_counts_body,
        out_type=jax.ShapeDtypeStruct((3, NW, N), jnp.float32),
        mesh=_mesh(),
        scratch_types=[
            pltpu.VMEM((EPW // 16, 16), jnp.int32),
            pltpu.VMEM((N,), jnp.float32),
        ],
        compiler_params=pltpu.CompilerParams(needs_layout_passes=False),
    )
    return kern(dst3)


def _sc_segsum_body(x_hbm, src_hbm, dst_hbm, dummy_hbm, out_hbm,
                    idx_s, idx_d, rows0, rows1, zbuf, acc, sem0, sem1):
    c = lax.axis_index("c")
    s = lax.axis_index("s")
    wid = s * NC + c
    row0 = s * RPT

    def _zrow(i, _):
        for j in range(D // 16):
            zbuf[i, pl.ds(j * 16, 16)] = jnp.zeros((16,), jnp.float32)
        return 0
    lax.fori_loop(0, ZR, _zrow, 0)

    def _zcopy(z, _):
        pltpu.sync_copy(zbuf, acc.at[pl.ds(row0 + z * ZR, ZR)])
        return 0
    lax.fori_loop(0, RPT // ZR, _zcopy, 0)
    plsc.subcore_barrier()

    def _start(j, buf, sem):
        pltpu.async_copy(x_hbm.at[idx_s.at[j]], buf, sem)

    def _wait(j, buf, sem):
        del j
        pltpu.make_async_copy(dummy_hbm, buf, sem).wait()

    def _scat(j, buf):
        pltpu.sync_copy(buf, acc.at[idx_d.at[j]], add=True)

    def _block(b, _):
        pltpu.sync_copy(src_hbm.at[wid * NBLK + b], idx_s)
        pltpu.sync_copy(dst_hbm.at[wid * NBLK + b], idx_d)
        _start(0, rows0, sem0)
        _start(1, rows1, sem1)
        _wait(0, rows0, sem0)
        _scat(0, rows0)

        def _pair(jj, _):
            jod = 1 + 2 * jj
            _start(jod + 1, rows0, sem0)
            _wait(jod, rows1, sem1)
            _scat(jod, rows1)

            @pl.when(jj < (BLK - 3) // 2)
            def _():
                _start(jod + 2, rows1, sem1)
            _wait(jod + 1, rows0, sem0)
            _scat(jod + 1, rows0)
            return 0
        lax.fori_loop(0, (BLK - 1) // 2, _pair, 0)
        return 0
    lax.fori_loop(0, NBLK, _block, 0)
    plsc.subcore_barrier()

    pltpu.sync_copy(acc.at[pl.ds(row0, RPT)], out_hbm.at[c, s])


@jax.jit
def _sc_segsum(x, src_r, dst_r):
    kern = pl.kernel(
        _sc_segsum_body,
        out_type=jax.ShapeDtypeStruct((NC, NS, RPT, D), jnp.float32),
        mesh=_mesh(),
        scratch_types=[
            pltpu.VMEM((BLK, K), jnp.int32),
            pltpu.VMEM((BLK, K), jnp.int32),
            pltpu.VMEM((K, D), jnp.float32),
            pltpu.VMEM((K, D), jnp.float32),
            pltpu.VMEM((ZR, D), jnp.float32),
            pltpu.VMEM_SHARED((N, D), jnp.float32),
            pltpu.SemaphoreType.DMA,
            pltpu.SemaphoreType.DMA,
        ],
    )
    dummy = jnp.zeros((K, D), jnp.float32)
    return kern(x, src_r, dst_r, dummy).reshape(NC, N, D)


BM = 400
GRID = N // BM


def _tc_a_body(s_ref, c1_ref, cg_ref, x_ref, w_ref, b_ref, o1_ref, o2_ref):
    ssum = s_ref[0] + s_ref[1]
    c1 = jnp.maximum(jnp.sum(c1_ref[...], axis=1), 1.0)
    m = ssum / c1[:, None]
    h = jnp.dot(m, w_ref[...], preferred_element_type=jnp.float32)
    e1 = jnp.maximum(h + b_ref[...], 0.0) + x_ref[...]
    cg = jnp.sum(cg_ref[...], axis=1) + 1.0
    dinv = lax.rsqrt(cg)
    o1_ref[...] = e1
    o2_ref[...] = e1 * dinv[:, None]


def _tc_b_body(s_ref, cg_ref, xs_ref, w_ref, b_ref, o_ref):
    ssum = s_ref[0] + s_ref[1]
    cg = jnp.sum(cg_ref[...], axis=1) + 1.0
    dinv = lax.rsqrt(cg)
    g = (ssum + xs_ref[...]) * dinv[:, None]
    o_ref[...] = jnp.dot(g, w_ref[...], preferred_element_type=jnp.float32) \
        + b_ref[...]


def _tc_c_body(s_ref, c3_ref, x_ref, w_ref, b_ref, o_ref):
    ssum = s_ref[0] + s_ref[1]
    c3 = jnp.maximum(jnp.sum(c3_ref[...], axis=1), 1.0)
    m = ssum / c3[:, None]
    h = jnp.dot(m, w_ref[...], preferred_element_type=jnp.float32)
    o_ref[...] = jnp.maximum(h + b_ref[...], 0.0) + x_ref[...]


def _row_spec(bm, d):
    return pl.BlockSpec((bm, d), lambda i: (i, 0))


_S_SPEC = pl.BlockSpec((NC, BM, D), lambda i: (0, i, 0))
_CNT_SPEC = pl.BlockSpec((BM, NW), lambda i: (i, 0))
_W_SPEC = pl.BlockSpec((D, D), lambda i: (0, 0))
_B_SPEC = pl.BlockSpec((1, D), lambda i: (0, 0))
_X_SPEC = _row_spec(BM, D)
_OUT = jax.ShapeDtypeStruct((N, D), jnp.float32)


@jax.jit
def _tc_stage_a(s_parts, c1_parts, cg_parts, x, w, b):
    return pl.pallas_call(
        _tc_a_body,
        grid=(GRID,),
        in_specs=[_S_SPEC, _CNT_SPEC, _CNT_SPEC, _X_SPEC, _W_SPEC, _B_SPEC],
        out_specs=[_X_SPEC, _X_SPEC],
        out_shape=[_OUT, _OUT],
    )(s_parts, c1_parts, cg_parts, x, w, b.reshape(1, D))


@jax.jit
def _tc_stage_b(s_parts, cg_parts, xs, w, b):
    return pl.pallas_call(
        _tc_b_body,
        grid=(GRID,),
        in_specs=[_S_SPEC, _CNT_SPEC, _X_SPEC, _W_SPEC, _B_SPEC],
        out_specs=_X_SPEC,
        out_shape=_OUT,
    )(s_parts, cg_parts, xs, w, b.reshape(1, D))


@jax.jit
def _tc_stage_c(s_parts, c3_parts, x, w, b):
    return pl.pallas_call(
        _tc_c_body,
        grid=(GRID,),
        in_specs=[_S_SPEC, _CNT_SPEC, _X_SPEC, _W_SPEC, _B_SPEC],
        out_specs=_X_SPEC,
        out_shape=_OUT,
    )(s_parts, c3_parts, x, w, b.reshape(1, D))


def kernel(embedding, down2up_path, same_level_edge_index, up2down_edge_index,
           W_d2u, b_d2u, W_gcn, b_gcn, W_u2d, b_u2d):
    e_d2u = down2up_path.astype(jnp.int32)
    e_gcn = same_level_edge_index.astype(jnp.int32)
    e_u2d = up2down_edge_index.astype(jnp.int32)

    def _split(e):
        return (e[0].reshape(NW * NBLK, BLK, K), e[1].reshape(NW * NBLK, BLK, K))

    src1, dst1 = _split(e_d2u)
    srcg, dstg = _split(e_gcn)
    src3, dst3 = _split(e_u2d)

    dsts = jnp.stack([
        e_d2u[1].reshape(NW, EPW // 16, 16),
        e_gcn[1].reshape(NW, EPW // 16, 16),
        e_u2d[1].reshape(NW, EPW // 16, 16),
    ])
    cnts = _sc_counts(dsts).transpose(0, 2, 1)

    s1 = _sc_segsum(embedding, src1, dst1)
    e1, xs = _tc_stage_a(s1, cnts[0], cnts[1], embedding, W_d2u, b_d2u)
    s2 = _sc_segsum(xs, srcg, dstg)
    e2 = _tc_stage_b(s2, cnts[1], xs, W_gcn, b_gcn)
    s3 = _sc_segsum(e2, src3, dst3)
    return _tc_stage_c(s3, cnts[2], e2, W_u2d, b_u2d)


# speedup vs baseline: 1.0179x; 1.0179x over previous
import functools

import jax
import jax.numpy as jnp
from jax import lax
from jax.experimental import pallas as pl
from jax.experimental.pallas import tpu as pltpu
from jax.experimental.pallas import tpu_sc as plsc

N = 10000
E = 320000
D = 128

NC = 2
NS = 16
NW = NC * NS
EPW = E // NW
K = 125
CH = EPW // K
BLK = 16
NBLK = CH // BLK
ZR = 25
RPT = N // NS

@functools.lru_cache(maxsize=None)
def _mesh():
    return plsc.VectorSubcoreMesh(
        core_axis_name="c", subcore_axis_name="s",
        num_cores=NC, num_subcores=NS)


def _sc_counts_body(dst3_hbm, out_hbm, idx_v, cnt_v):
    c = lax.axis_index("c")
    s = lax.axis_index("s")
    wid = s * NC + c
    for set_i in range(3):
        pltpu.sync_copy(dst3_hbm.at[set_i, wid], idx_v)

        def _zero(i, _):
            for u in range(5):
                cnt_v[pl.ds((i * 5 + u) * 16, 16)] = jnp.zeros(
                    (16,), jnp.float32)
            return 0
        lax.fori_loop(0, N // 80, _zero, 0)

        def _count(i, _):
            idx = idx_v[i]
            plsc.addupdate_scatter(cnt_v, [idx], jnp.ones((16,), jnp.float32))
            return 0
        lax.fori_loop(0, EPW // 16, _count, 0)
        pltpu.sync_copy(cnt_v, out_hbm.at[set_i, wid])


@jax.jit
def _sc_counts(dst3):
    kern = pl.kernel(
        _sc_counts_body,
        out_type=jax.ShapeDtypeStruct((3, NW, N), jnp.float32),
        mesh=_mesh(),
        scratch_types=[
            pltpu.VMEM((EPW // 16, 16), jnp.int32),
            pltpu.VMEM((N,), jnp.float32),
        ],
        compiler_params=pltpu.CompilerParams(needs_layout_passes=False),
    )
    return kern(dst3)


def _sc_segsum_body(x_hbm, src_hbm, dst_hbm, dummy_hbm, out_hbm,
                    idx_s, idx_d, rows0, rows1, zbuf, acc, sem0, sem1):
    c = lax.axis_index("c")
    s = lax.axis_index("s")
    wid = s * NC + c
    row0 = s * RPT

    def _zrow(i, _):
        for j in range(D // 16):
            zbuf[i, pl.ds(j * 16, 16)] = jnp.zeros((16,), jnp.float32)
        return 0
    lax.fori_loop(0, ZR, _zrow, 0)

    def _zcopy(z, _):
        pltpu.sync_copy(zbuf, acc.at[pl.ds(row0 + z * ZR, ZR)])
        return 0
    lax.fori_loop(0, RPT // ZR, _zcopy, 0)
    plsc.subcore_barrier()

    def _start(j, buf, sem):
        pltpu.async_copy(x_hbm.at[idx_s.at[j]], buf, sem)

    def _wait(j, buf, sem):
        del j
        pltpu.make_async_copy(dummy_hbm, buf, sem).wait()

    def _scat(j, buf):
        pltpu.sync_copy(buf, acc.at[idx_d.at[j]], add=True)

    def _block(b, _):
        pltpu.sync_copy(src_hbm.at[wid * NBLK + b], idx_s)
        pltpu.sync_copy(dst_hbm.at[wid * NBLK + b], idx_d)
        _start(0, rows0, sem0)
        _start(1, rows1, sem1)
        _wait(0, rows0, sem0)
        _scat(0, rows0)

        def _pair(jj, _):
            jod = 1 + 2 * jj
            _start(jod + 1, rows0, sem0)
            _wait(jod, rows1, sem1)
            _scat(jod, rows1)
            if BLK % 2:
                @pl.when(jj < (BLK - 3) // 2)
                def _():
                    _start(jod + 2, rows1, sem1)
            else:
                _start(jod + 2, rows1, sem1)
            _wait(jod + 1, rows0, sem0)
            _scat(jod + 1, rows0)
            return 0
        lax.fori_loop(0, (BLK - 1) // 2 if BLK % 2 else (BLK - 2) // 2,
                      _pair, 0)
        if BLK % 2 == 0:
            _wait(BLK - 1, rows1, sem1)
            _scat(BLK - 1, rows1)
        return 0
    lax.fori_loop(0, NBLK, _block, 0)
    plsc.subcore_barrier()

    pltpu.sync_copy(acc.at[pl.ds(row0, RPT)], out_hbm.at[c, s])


@jax.jit
def _sc_segsum(x, src_r, dst_r):
    kern = pl.kernel(
        _sc_segsum_body,
        out_type=jax.ShapeDtypeStruct((NC, NS, RPT, D), jnp.float32),
        mesh=_mesh(),
        scratch_types=[
            pltpu.VMEM((BLK, K), jnp.int32),
            pltpu.VMEM((BLK, K), jnp.int32),
            pltpu.VMEM((K, D), jnp.float32),
            pltpu.VMEM((K, D), jnp.float32),
            pltpu.VMEM((ZR, D), jnp.float32),
            pltpu.VMEM_SHARED((N, D), jnp.float32),
            pltpu.SemaphoreType.DMA,
            pltpu.SemaphoreType.DMA,
        ],
    )
    dummy = jnp.zeros((K, D), jnp.float32)
    return kern(x, src_r, dst_r, dummy).reshape(NC, N, D)


BM = 400
GRID = N // BM


def _tc_a_body(s_ref, c1_ref, cg_ref, x_ref, w_ref, b_ref, o_ref):
    ssum = s_ref[0] + s_ref[1]
    c1 = jnp.maximum(jnp.sum(c1_ref[...], axis=1), 1.0)
    m = ssum / c1[:, None]
    h = jnp.dot(m, w_ref[...], preferred_element_type=jnp.float32)
    e1 = jnp.maximum(h + b_ref[...], 0.0) + x_ref[...]
    cg = jnp.sum(cg_ref[...], axis=1) + 1.0
    dinv = lax.rsqrt(cg)
    o_ref[...] = e1 * dinv[:, None]


def _tc_b_body(s_ref, cg_ref, xs_ref, w_ref, b_ref, o_ref):
    ssum = s_ref[0] + s_ref[1]
    cg = jnp.sum(cg_ref[...], axis=1) + 1.0
    dinv = lax.rsqrt(cg)
    g = (ssum + xs_ref[...]) * dinv[:, None]
    o_ref[...] = jnp.dot(g, w_ref[...], preferred_element_type=jnp.float32) \
        + b_ref[...]


def _tc_c_body(s_ref, c3_ref, x_ref, w_ref, b_ref, o_ref):
    ssum = s_ref[0] + s_ref[1]
    c3 = jnp.maximum(jnp.sum(c3_ref[...], axis=1), 1.0)
    m = ssum / c3[:, None]
    h = jnp.dot(m, w_ref[...], preferred_element_type=jnp.float32)
    o_ref[...] = jnp.maximum(h + b_ref[...], 0.0) + x_ref[...]


def _row_spec(bm, d):
    return pl.BlockSpec((bm, d), lambda i: (i, 0))


_S_SPEC = pl.BlockSpec((NC, BM, D), lambda i: (0, i, 0))
_CNT_SPEC = pl.BlockSpec((BM, NW), lambda i: (i, 0))
_W_SPEC = pl.BlockSpec((D, D), lambda i: (0, 0))
_B_SPEC = pl.BlockSpec((1, D), lambda i: (0, 0))
_X_SPEC = _row_spec(BM, D)
_OUT = jax.ShapeDtypeStruct((N, D), jnp.float32)


@jax.jit
def _tc_stage_a(s_parts, c1_parts, cg_parts, x, w, b):
    return pl.pallas_call(
        _tc_a_body,
        grid=(GRID,),
        in_specs=[_S_SPEC, _CNT_SPEC, _CNT_SPEC, _X_SPEC, _W_SPEC, _B_SPEC],
        out_specs=_X_SPEC,
        out_shape=_OUT,
    )(s_parts, c1_parts, cg_parts, x, w, b.reshape(1, D))


@jax.jit
def _tc_stage_b(s_parts, cg_parts, xs, w, b):
    return pl.pallas_call(
        _tc_b_body,
        grid=(GRID,),
        in_specs=[_S_SPEC, _CNT_SPEC, _X_SPEC, _W_SPEC, _B_SPEC],
        out_specs=_X_SPEC,
        out_shape=_OUT,
    )(s_parts, cg_parts, xs, w, b.reshape(1, D))


@jax.jit
def _tc_stage_c(s_parts, c3_parts, x, w, b):
    return pl.pallas_call(
        _tc_c_body,
        grid=(GRID,),
        in_specs=[_S_SPEC, _CNT_SPEC, _X_SPEC, _W_SPEC, _B_SPEC],
        out_specs=_X_SPEC,
        out_shape=_OUT,
    )(s_parts, c3_parts, x, w, b.reshape(1, D))


def kernel(embedding, down2up_path, same_level_edge_index, up2down_edge_index,
           W_d2u, b_d2u, W_gcn, b_gcn, W_u2d, b_u2d):
    e_d2u = down2up_path.astype(jnp.int32)
    e_gcn = same_level_edge_index.astype(jnp.int32)
    e_u2d = up2down_edge_index.astype(jnp.int32)

    def _split(e):
        return (e[0].reshape(NW * NBLK, BLK, K), e[1].reshape(NW * NBLK, BLK, K))

    src1, dst1 = _split(e_d2u)
    srcg, dstg = _split(e_gcn)
    src3, dst3 = _split(e_u2d)

    dsts = jnp.stack([
        e_d2u[1].reshape(NW, EPW // 16, 16),
        e_gcn[1].reshape(NW, EPW // 16, 16),
        e_u2d[1].reshape(NW, EPW // 16, 16),
    ])
    cnts = _sc_counts(dsts).transpose(0, 2, 1)

    s1 = _sc_segsum(embedding, src1, dst1)
    xs = _tc_stage_a(s1, cnts[0], cnts[1], embedding, W_d2u, b_d2u)
    s2 = _sc_segsum(xs, srcg, dstg)
    e2 = _tc_stage_b(s2, cnts[1], xs, W_gcn, b_gcn)
    s3 = _sc_segsum(e2, src3, dst3)
    return _tc_stage_c(s3, cnts[2], e2, W_u2d, b_u2d)


# speedup vs baseline: 1.1099x; 1.0904x over previous
import functools

import jax
import jax.numpy as jnp
from jax import lax
from jax.experimental import pallas as pl
from jax.experimental.pallas import tpu as pltpu
from jax.experimental.pallas import tpu_sc as plsc

N = 10000
E = 320000
D = 128

NC = 2
NS = 16
NW = NC * NS
EPW = E // NW
K = 125
CH = EPW // K
BLK = 16
NBLK = CH // BLK
ZR = 25
RPT = N // NS

@functools.lru_cache(maxsize=None)
def _mesh():
    return plsc.VectorSubcoreMesh(
        core_axis_name="c", subcore_axis_name="s",
        num_cores=NC, num_subcores=NS)


def _sc_counts_body(dst0_hbm, dst1_hbm, dst2_hbm, out_hbm, idx_v, cnt_v):
    c = lax.axis_index("c")
    s = lax.axis_index("s")
    wid = s * NC + c
    for set_i, dst_hbm in enumerate((dst0_hbm, dst1_hbm, dst2_hbm)):
        pltpu.sync_copy(dst_hbm.at[wid], idx_v)

        def _zero(i, _):
            for u in range(5):
                cnt_v[pl.ds((i * 5 + u) * 16, 16)] = jnp.zeros(
                    (16,), jnp.float32)
            return 0
        lax.fori_loop(0, N // 80, _zero, 0)

        def _count(i, _):
            idx = idx_v[i]
            plsc.addupdate_scatter(cnt_v, [idx], jnp.ones((16,), jnp.float32))
            return 0
        lax.fori_loop(0, EPW // 16, _count, 0)
        pltpu.sync_copy(cnt_v, out_hbm.at[set_i, wid])


@jax.jit
def _sc_counts(dst0, dst1, dst2):
    kern = pl.kernel(
        _sc_counts_body,
        out_type=jax.ShapeDtypeStruct((3, NW, N), jnp.float32),
        mesh=_mesh(),
        scratch_types=[
            pltpu.VMEM((EPW // 16, 16), jnp.int32),
            pltpu.VMEM((N,), jnp.float32),
        ],
        compiler_params=pltpu.CompilerParams(needs_layout_passes=False),
    )
    return kern(dst0, dst1, dst2)


def _sc_segsum_body(x_hbm, src_hbm, dst_hbm, dummy_hbm, out_hbm,
                    idx_s, idx_d, rows0, rows1, zbuf, acc, sem0, sem1):
    c = lax.axis_index("c")
    s = lax.axis_index("s")
    wid = s * NC + c
    row0 = s * RPT

    def _zrow(i, _):
        for j in range(D // 16):
            zbuf[i, pl.ds(j * 16, 16)] = jnp.zeros((16,), jnp.float32)
        return 0
    lax.fori_loop(0, ZR, _zrow, 0)

    def _zcopy(z, _):
        pltpu.sync_copy(zbuf, acc.at[pl.ds(row0 + z * ZR, ZR)])
        return 0
    lax.fori_loop(0, RPT // ZR, _zcopy, 0)
    plsc.subcore_barrier()

    def _start(j, buf, sem):
        pltpu.async_copy(x_hbm.at[idx_s.at[j]], buf, sem)

    def _wait(j, buf, sem):
        del j
        pltpu.make_async_copy(dummy_hbm, buf, sem).wait()

    def _scat(j, buf):
        pltpu.sync_copy(buf, acc.at[idx_d.at[j]], add=True)

    def _block(b, _):
        pltpu.sync_copy(src_hbm.at[wid * NBLK + b], idx_s)
        pltpu.sync_copy(dst_hbm.at[wid * NBLK + b], idx_d)
        _start(0, rows0, sem0)
        _start(1, rows1, sem1)
        _wait(0, rows0, sem0)
        _scat(0, rows0)

        def _pair(jj, _):
            jod = 1 + 2 * jj
            _start(jod + 1, rows0, sem0)
            _wait(jod, rows1, sem1)
            _scat(jod, rows1)
            if BLK % 2:
                @pl.when(jj < (BLK - 3) // 2)
                def _():
                    _start(jod + 2, rows1, sem1)
            else:
                _start(jod + 2, rows1, sem1)
            _wait(jod + 1, rows0, sem0)
            _scat(jod + 1, rows0)
            return 0
        lax.fori_loop(0, (BLK - 1) // 2 if BLK % 2 else (BLK - 2) // 2,
                      _pair, 0)
        if BLK % 2 == 0:
            _wait(BLK - 1, rows1, sem1)
            _scat(BLK - 1, rows1)
        return 0
    lax.fori_loop(0, NBLK, _block, 0)
    plsc.subcore_barrier()

    pltpu.sync_copy(acc.at[pl.ds(row0, RPT)], out_hbm.at[c, s])


@jax.jit
def _sc_segsum(x, src_r, dst_r):
    kern = pl.kernel(
        _sc_segsum_body,
        out_type=jax.ShapeDtypeStruct((NC, NS, RPT, D), jnp.float32),
        mesh=_mesh(),
        scratch_types=[
            pltpu.VMEM((BLK, K), jnp.int32),
            pltpu.VMEM((BLK, K), jnp.int32),
            pltpu.VMEM((K, D), jnp.float32),
            pltpu.VMEM((K, D), jnp.float32),
            pltpu.VMEM((ZR, D), jnp.float32),
            pltpu.VMEM_SHARED((N, D), jnp.float32),
            pltpu.SemaphoreType.DMA,
            pltpu.SemaphoreType.DMA,
        ],
    )
    dummy = jnp.zeros((K, D), jnp.float32)
    return kern(x, src_r, dst_r, dummy).reshape(NC, N, D)


BM = 1000
GRID = N // BM


def _tc_a_body(s_ref, c1_ref, cg_ref, x_ref, w_ref, b_ref, o_ref):
    ssum = s_ref[0] + s_ref[1]
    c1 = jnp.maximum(jnp.sum(c1_ref[...], axis=1), 1.0)
    m = ssum / c1[:, None]
    h = jnp.dot(m, w_ref[...], preferred_element_type=jnp.float32)
    e1 = jnp.maximum(h + b_ref[...], 0.0) + x_ref[...]
    cg = jnp.sum(cg_ref[...], axis=1) + 1.0
    dinv = lax.rsqrt(cg)
    o_ref[...] = e1 * dinv[:, None]


def _tc_b_body(s_ref, cg_ref, xs_ref, w_ref, b_ref, o_ref):
    ssum = s_ref[0] + s_ref[1]
    cg = jnp.sum(cg_ref[...], axis=1) + 1.0
    dinv = lax.rsqrt(cg)
    g = (ssum + xs_ref[...]) * dinv[:, None]
    o_ref[...] = jnp.dot(g, w_ref[...], preferred_element_type=jnp.float32) \
        + b_ref[...]


def _tc_c_body(s_ref, c3_ref, x_ref, w_ref, b_ref, o_ref):
    ssum = s_ref[0] + s_ref[1]
    c3 = jnp.maximum(jnp.sum(c3_ref[...], axis=1), 1.0)
    m = ssum / c3[:, None]
    h = jnp.dot(m, w_ref[...], preferred_element_type=jnp.float32)
    o_ref[...] = jnp.maximum(h + b_ref[...], 0.0) + x_ref[...]


def _row_spec(bm, d):
    return pl.BlockSpec((bm, d), lambda i: (i, 0))


_S_SPEC = pl.BlockSpec((NC, BM, D), lambda i: (0, i, 0))
_CNT_SPEC = pl.BlockSpec((BM, NW), lambda i: (i, 0))
_W_SPEC = pl.BlockSpec((D, D), lambda i: (0, 0))
_B_SPEC = pl.BlockSpec((1, D), lambda i: (0, 0))
_X_SPEC = _row_spec(BM, D)
_OUT = jax.ShapeDtypeStruct((N, D), jnp.float32)


@jax.jit
def _tc_stage_a(s_parts, c1_parts, cg_parts, x, w, b):
    return pl.pallas_call(
        _tc_a_body,
        grid=(GRID,),
        compiler_params=pltpu.CompilerParams(
            dimension_semantics=("parallel",)),
        in_specs=[_S_SPEC, _CNT_SPEC, _CNT_SPEC, _X_SPEC, _W_SPEC, _B_SPEC],
        out_specs=_X_SPEC,
        out_shape=_OUT,
    )(s_parts, c1_parts, cg_parts, x, w, b.reshape(1, D))


@jax.jit
def _tc_stage_b(s_parts, cg_parts, xs, w, b):
    return pl.pallas_call(
        _tc_b_body,
        grid=(GRID,),
        compiler_params=pltpu.CompilerParams(
            dimension_semantics=("parallel",)),
        in_specs=[_S_SPEC, _CNT_SPEC, _X_SPEC, _W_SPEC, _B_SPEC],
        out_specs=_X_SPEC,
        out_shape=_OUT,
    )(s_parts, cg_parts, xs, w, b.reshape(1, D))


@jax.jit
def _tc_stage_c(s_parts, c3_parts, x, w, b):
    return pl.pallas_call(
        _tc_c_body,
        grid=(GRID,),
        compiler_params=pltpu.CompilerParams(
            dimension_semantics=("parallel",)),
        in_specs=[_S_SPEC, _CNT_SPEC, _X_SPEC, _W_SPEC, _B_SPEC],
        out_specs=_X_SPEC,
        out_shape=_OUT,
    )(s_parts, c3_parts, x, w, b.reshape(1, D))


def kernel(embedding, down2up_path, same_level_edge_index, up2down_edge_index,
           W_d2u, b_d2u, W_gcn, b_gcn, W_u2d, b_u2d):
    e_d2u = down2up_path.astype(jnp.int32)
    e_gcn = same_level_edge_index.astype(jnp.int32)
    e_u2d = up2down_edge_index.astype(jnp.int32)

    def _split(e):
        return (e[0].reshape(NW * NBLK, BLK, K), e[1].reshape(NW * NBLK, BLK, K))

    src1, dst1 = _split(e_d2u)
    srcg, dstg = _split(e_gcn)
    src3, dst3 = _split(e_u2d)

    cnts = _sc_counts(
        e_d2u[1].reshape(NW, EPW // 16, 16),
        e_gcn[1].reshape(NW, EPW // 16, 16),
        e_u2d[1].reshape(NW, EPW // 16, 16),
    ).transpose(0, 2, 1)

    s1 = _sc_segsum(embedding, src1, dst1)
    xs = _tc_stage_a(s1, cnts[0], cnts[1], embedding, W_d2u, b_d2u)
    s2 = _sc_segsum(xs, srcg, dstg)
    e2 = _tc_stage_b(s2, cnts[1], xs, W_gcn, b_gcn)
    s3 = _sc_segsum(e2, src3, dst3)
    return _tc_stage_c(s3, cnts[2], e2, W_u2d, b_u2d)


# speedup vs baseline: 1.1397x; 1.0268x over previous
import functools

import jax
import jax.numpy as jnp
from jax import lax
from jax.experimental import pallas as pl
from jax.experimental.pallas import tpu as pltpu
from jax.experimental.pallas import tpu_sc as plsc

N = 10000
E = 320000
D = 128

NC = 2
NS = 16
NW = NC * NS
EPW = E // NW
K = 125
CH = EPW // K
BLK = 16
NBLK = CH // BLK
ZR = 25
RPT = N // NS

@functools.lru_cache(maxsize=None)
def _mesh():
    return plsc.VectorSubcoreMesh(
        core_axis_name="c", subcore_axis_name="s",
        num_cores=NC, num_subcores=NS)


def _sc_counts_body(dst0_hbm, dst1_hbm, dst2_hbm, out_hbm, idx_v, cnt_v):
    c = lax.axis_index("c")
    s = lax.axis_index("s")
    wid = s * NC + c
    for set_i, dst_hbm in enumerate((dst0_hbm, dst1_hbm, dst2_hbm)):
        pltpu.sync_copy(dst_hbm.at[wid], idx_v)

        def _zero(i, _):
            for u in range(5):
                cnt_v[pl.ds((i * 5 + u) * 16, 16)] = jnp.zeros(
                    (16,), jnp.float32)
            return 0
        lax.fori_loop(0, N // 80, _zero, 0)

        def _count(i, _):
            idx = idx_v[i]
            plsc.addupdate_scatter(cnt_v, [idx], jnp.ones((16,), jnp.float32))
            return 0
        lax.fori_loop(0, EPW // 16, _count, 0)
        pltpu.sync_copy(cnt_v, out_hbm.at[set_i, wid])


@jax.jit
def _sc_counts(dst0, dst1, dst2):
    kern = pl.kernel(
        _sc_counts_body,
        out_type=jax.ShapeDtypeStruct((3, NW, N), jnp.float32),
        mesh=_mesh(),
        scratch_types=[
            pltpu.VMEM((EPW // 16, 16), jnp.int32),
            pltpu.VMEM((N,), jnp.float32),
        ],
        compiler_params=pltpu.CompilerParams(needs_layout_passes=False),
    )
    return kern(dst0, dst1, dst2)


def _sc_segsum_body(x_hbm, src_hbm, dst_hbm, dummy_hbm, out_hbm,
                    idx_s0, idx_d0, idx_s1, idx_d1, rows0, rows1, zbuf, acc,
                    sem0, sem1, sem_i0, sem_i1):
    c = lax.axis_index("c")
    s = lax.axis_index("s")
    wid = s * NC + c
    row0 = s * RPT

    def _zrow(i, _):
        for j in range(D // 16):
            zbuf[i, pl.ds(j * 16, 16)] = jnp.zeros((16,), jnp.float32)
        return 0
    lax.fori_loop(0, ZR, _zrow, 0)

    def _zcopy(z, _):
        pltpu.sync_copy(zbuf, acc.at[pl.ds(row0 + z * ZR, ZR)])
        return 0
    lax.fori_loop(0, RPT // ZR, _zcopy, 0)
    plsc.subcore_barrier()

    def _start(idx, j, buf, sem):
        pltpu.async_copy(x_hbm.at[idx.at[j]], buf, sem)

    def _wait(buf, sem):
        pltpu.make_async_copy(dummy_hbm, buf, sem).wait()

    def _scat(idx, j, buf):
        pltpu.sync_copy(buf, acc.at[idx.at[j]], add=True)

    def _stage(b, is_, id_, sem):
        pltpu.async_copy(src_hbm.at[wid * NBLK + b], is_, sem)
        pltpu.async_copy(dst_hbm.at[wid * NBLK + b], id_, sem)

    def _stage_wait(is_, id_, sem):
        pltpu.make_async_copy(src_hbm.at[0], is_, sem).wait()
        pltpu.make_async_copy(src_hbm.at[0], id_, sem).wait()

    def _run_block(is_, id_):
        _start(is_, 0, rows0, sem0)
        _start(is_, 1, rows1, sem1)
        _wait(rows0, sem0)
        _scat(id_, 0, rows0)

        def _pair(jj, _):
            jod = 1 + 2 * jj
            _start(is_, jod + 1, rows0, sem0)
            _wait(rows1, sem1)
            _scat(id_, jod, rows1)
            if BLK % 2:
                @pl.when(jj < (BLK - 3) // 2)
                def _():
                    _start(is_, jod + 2, rows1, sem1)
            else:
                _start(is_, jod + 2, rows1, sem1)
            _wait(rows0, sem0)
            _scat(id_, jod + 1, rows0)
            return 0
        lax.fori_loop(0, (BLK - 1) // 2 if BLK % 2 else (BLK - 2) // 2,
                      _pair, 0)
        if BLK % 2 == 0:
            _wait(rows1, sem1)
            _scat(id_, BLK - 1, rows1)

    bufs = ((idx_s0, idx_d0, sem_i0), (idx_s1, idx_d1, sem_i1))
    _stage(0, idx_s0, idx_d0, sem_i0)
    _stage_wait(idx_s0, idx_d0, sem_i0)
    for b in range(NBLK):
        cur_s, cur_d, _cur_sem = bufs[b % 2]
        nxt_s, nxt_d, nxt_sem = bufs[(b + 1) % 2]
        if b + 1 < NBLK:
            _stage(b + 1, nxt_s, nxt_d, nxt_sem)
        _run_block(cur_s, cur_d)
        if b + 1 < NBLK:
            _stage_wait(nxt_s, nxt_d, nxt_sem)
    plsc.subcore_barrier()

    pltpu.sync_copy(acc.at[pl.ds(row0, RPT)], out_hbm.at[c, s])


@jax.jit
def _sc_segsum(x, src_r, dst_r):
    kern = pl.kernel(
        _sc_segsum_body,
        out_type=jax.ShapeDtypeStruct((NC, NS, RPT, D), jnp.float32),
        mesh=_mesh(),
        scratch_types=[
            pltpu.VMEM((BLK, K), jnp.int32),
            pltpu.VMEM((BLK, K), jnp.int32),
            pltpu.VMEM((BLK, K), jnp.int32),
            pltpu.VMEM((BLK, K), jnp.int32),
            pltpu.VMEM((K, D), jnp.float32),
            pltpu.VMEM((K, D), jnp.float32),
            pltpu.VMEM((ZR, D), jnp.float32),
            pltpu.VMEM_SHARED((N, D), jnp.float32),
            pltpu.SemaphoreType.DMA,
            pltpu.SemaphoreType.DMA,
            pltpu.SemaphoreType.DMA,
            pltpu.SemaphoreType.DMA,
        ],
    )
    dummy = jnp.zeros((K, D), jnp.float32)
    return kern(x, src_r, dst_r, dummy).reshape(NC, N, D)


BM = 1000
GRID = N // BM


def _tc_a_body(s_ref, c1_ref, cg_ref, x_ref, w_ref, b_ref, o_ref):
    ssum = s_ref[0] + s_ref[1]
    c1 = jnp.maximum(jnp.sum(c1_ref[...], axis=1), 1.0)
    m = ssum / c1[:, None]
    h = jnp.dot(m, w_ref[...], preferred_element_type=jnp.float32)
    e1 = jnp.maximum(h + b_ref[...], 0.0) + x_ref[...]
    cg = jnp.sum(cg_ref[...], axis=1) + 1.0
    dinv = lax.rsqrt(cg)
    o_ref[...] = e1 * dinv[:, None]


def _tc_b_body(s_ref, cg_ref, xs_ref, w_ref, b_ref, o_ref):
    ssum = s_ref[0] + s_ref[1]
    cg = jnp.sum(cg_ref[...], axis=1) + 1.0
    dinv = lax.rsqrt(cg)
    g = (ssum + xs_ref[...]) * dinv[:, None]
    o_ref[...] = jnp.dot(g, w_ref[...], preferred_element_type=jnp.float32) \
        + b_ref[...]


def _tc_c_body(s_ref, c3_ref, x_ref, w_ref, b_ref, o_ref):
    ssum = s_ref[0] + s_ref[1]
    c3 = jnp.maximum(jnp.sum(c3_ref[...], axis=1), 1.0)
    m = ssum / c3[:, None]
    h = jnp.dot(m, w_ref[...], preferred_element_type=jnp.float32)
    o_ref[...] = jnp.maximum(h + b_ref[...], 0.0) + x_ref[...]


def _row_spec(bm, d):
    return pl.BlockSpec((bm, d), lambda i: (i, 0))


_S_SPEC = pl.BlockSpec((NC, BM, D), lambda i: (0, i, 0))
_CNT_SPEC = pl.BlockSpec((BM, NW), lambda i: (i, 0))
_W_SPEC = pl.BlockSpec((D, D), lambda i: (0, 0))
_B_SPEC = pl.BlockSpec((1, D), lambda i: (0, 0))
_X_SPEC = _row_spec(BM, D)
_OUT = jax.ShapeDtypeStruct((N, D), jnp.float32)


@jax.jit
def _tc_stage_a(s_parts, c1_parts, cg_parts, x, w, b):
    return pl.pallas_call(
        _tc_a_body,
        grid=(GRID,),
        compiler_params=pltpu.CompilerParams(
            dimension_semantics=("parallel",)),
        in_specs=[_S_SPEC, _CNT_SPEC, _CNT_SPEC, _X_SPEC, _W_SPEC, _B_SPEC],
        out_specs=_X_SPEC,
        out_shape=_OUT,
    )(s_parts, c1_parts, cg_parts, x, w, b.reshape(1, D))


@jax.jit
def _tc_stage_b(s_parts, cg_parts, xs, w, b):
    return pl.pallas_call(
        _tc_b_body,
        grid=(GRID,),
        compiler_params=pltpu.CompilerParams(
            dimension_semantics=("parallel",)),
        in_specs=[_S_SPEC, _CNT_SPEC, _X_SPEC, _W_SPEC, _B_SPEC],
        out_specs=_X_SPEC,
        out_shape=_OUT,
    )(s_parts, cg_parts, xs, w, b.reshape(1, D))


@jax.jit
def _tc_stage_c(s_parts, c3_parts, x, w, b):
    return pl.pallas_call(
        _tc_c_body,
        grid=(GRID,),
        compiler_params=pltpu.CompilerParams(
            dimension_semantics=("parallel",)),
        in_specs=[_S_SPEC, _CNT_SPEC, _X_SPEC, _W_SPEC, _B_SPEC],
        out_specs=_X_SPEC,
        out_shape=_OUT,
    )(s_parts, c3_parts, x, w, b.reshape(1, D))


def kernel(embedding, down2up_path, same_level_edge_index, up2down_edge_index,
           W_d2u, b_d2u, W_gcn, b_gcn, W_u2d, b_u2d):
    e_d2u = down2up_path.astype(jnp.int32)
    e_gcn = same_level_edge_index.astype(jnp.int32)
    e_u2d = up2down_edge_index.astype(jnp.int32)

    def _split(e):
        return (e[0].reshape(NW * NBLK, BLK, K), e[1].reshape(NW * NBLK, BLK, K))

    src1, dst1 = _split(e_d2u)
    srcg, dstg = _split(e_gcn)
    src3, dst3 = _split(e_u2d)

    cnts = _sc_counts(
        e_d2u[1].reshape(NW, EPW // 16, 16),
        e_gcn[1].reshape(NW, EPW // 16, 16),
        e_u2d[1].reshape(NW, EPW // 16, 16),
    ).transpose(0, 2, 1)

    s1 = _sc_segsum(embedding, src1, dst1)
    xs = _tc_stage_a(s1, cnts[0], cnts[1], embedding, W_d2u, b_d2u)
    s2 = _sc_segsum(xs, srcg, dstg)
    e2 = _tc_stage_b(s2, cnts[1], xs, W_gcn, b_gcn)
    s3 = _sc_segsum(e2, src3, dst3)
    return _tc_stage_c(s3, cnts[2], e2, W_u2d, b_u2d)


# speedup vs baseline: 1.1554x; 1.0138x over previous
import functools

import jax
import jax.numpy as jnp
from jax import lax
from jax.experimental import pallas as pl
from jax.experimental.pallas import tpu as pltpu
from jax.experimental.pallas import tpu_sc as plsc

N = 10000
E = 320000
D = 128

NC = 2
NS = 16
NW = NC * NS
EPW = E // NW
K = 125
CH = EPW // K
BLK = 16
NBLK = CH // BLK
ZR = 25
RPT = N // NS

@functools.lru_cache(maxsize=None)
def _mesh():
    return plsc.VectorSubcoreMesh(
        core_axis_name="c", subcore_axis_name="s",
        num_cores=NC, num_subcores=NS)


def _sc_counts_body(dst0_hbm, dst1_hbm, dst2_hbm, out_hbm, idx_v, cnt_v):
    c = lax.axis_index("c")
    s = lax.axis_index("s")
    wid = s * NC + c
    for set_i, dst_hbm in enumerate((dst0_hbm, dst1_hbm, dst2_hbm)):
        pltpu.sync_copy(dst_hbm.at[wid], idx_v)

        def _zero(i, _):
            for u in range(5):
                cnt_v[pl.ds((i * 5 + u) * 16, 16)] = jnp.zeros(
                    (16,), jnp.float32)
            return 0
        lax.fori_loop(0, N // 80, _zero, 0)

        def _count(i, _):
            idx = idx_v[i]
            plsc.addupdate_scatter(cnt_v, [idx], jnp.ones((16,), jnp.float32))
            return 0
        lax.fori_loop(0, EPW // 16, _count, 0)
        pltpu.sync_copy(cnt_v, out_hbm.at[set_i, wid])


@jax.jit
def _sc_counts(dst0, dst1, dst2):
    kern = pl.kernel(
        _sc_counts_body,
        out_type=jax.ShapeDtypeStruct((3, NW, N), jnp.float32),
        mesh=_mesh(),
        scratch_types=[
            pltpu.VMEM((EPW // 16, 16), jnp.int32),
            pltpu.VMEM((N,), jnp.float32),
        ],
        compiler_params=pltpu.CompilerParams(needs_layout_passes=False),
    )
    return kern(dst0, dst1, dst2)


def _sc_segsum_body(x_hbm, src_hbm, dst_hbm, dummy_hbm, out_hbm,
                    idx_s0, idx_d0, idx_s1, idx_d1, rows0, rows1, zbuf, acc,
                    sem0, sem1, sem_i0, sem_i1):
    c = lax.axis_index("c")
    s = lax.axis_index("s")
    wid = s * NC + c
    row0 = s * RPT

    def _zrow(i, _):
        for j in range(D // 16):
            zbuf[i, pl.ds(j * 16, 16)] = jnp.zeros((16,), jnp.float32)
        return 0
    lax.fori_loop(0, ZR, _zrow, 0)

    def _zcopy(z, _):
        pltpu.sync_copy(zbuf, acc.at[pl.ds(row0 + z * ZR, ZR)])
        return 0
    lax.fori_loop(0, RPT // ZR, _zcopy, 0)
    plsc.subcore_barrier()

    def _start(idx, j, buf, sem):
        pltpu.async_copy(x_hbm.at[idx.at[j]], buf, sem)

    def _wait(buf, sem):
        pltpu.make_async_copy(dummy_hbm, buf, sem).wait()

    def _scat(idx, j, buf):
        pltpu.sync_copy(buf, acc.at[idx.at[j]], add=True)

    def _stage(b, is_, id_, sem):
        pltpu.async_copy(src_hbm.at[wid * NBLK + b], is_, sem)
        pltpu.async_copy(dst_hbm.at[wid * NBLK + b], id_, sem)

    def _stage_wait(is_, id_, sem):
        pltpu.make_async_copy(src_hbm.at[0], is_, sem).wait()
        pltpu.make_async_copy(src_hbm.at[0], id_, sem).wait()

    def _run_block(is_, id_):
        _start(is_, 0, rows0, sem0)
        _start(is_, 1, rows1, sem1)
        _wait(rows0, sem0)
        _scat(id_, 0, rows0)

        def _pair(jj, _):
            jod = 1 + 2 * jj
            _start(is_, jod + 1, rows0, sem0)
            _wait(rows1, sem1)
            _scat(id_, jod, rows1)
            if BLK % 2:
                @pl.when(jj < (BLK - 3) // 2)
                def _():
                    _start(is_, jod + 2, rows1, sem1)
            else:
                _start(is_, jod + 2, rows1, sem1)
            _wait(rows0, sem0)
            _scat(id_, jod + 1, rows0)
            return 0
        lax.fori_loop(0, (BLK - 1) // 2 if BLK % 2 else (BLK - 2) // 2,
                      _pair, 0)
        if BLK % 2 == 0:
            _wait(rows1, sem1)
            _scat(id_, BLK - 1, rows1)

    bufs = ((idx_s0, idx_d0, sem_i0), (idx_s1, idx_d1, sem_i1))
    _stage(0, idx_s0, idx_d0, sem_i0)
    _stage_wait(idx_s0, idx_d0, sem_i0)
    for b in range(NBLK):
        cur_s, cur_d, _cur_sem = bufs[b % 2]
        nxt_s, nxt_d, nxt_sem = bufs[(b + 1) % 2]
        if b + 1 < NBLK:
            _stage(b + 1, nxt_s, nxt_d, nxt_sem)
        _run_block(cur_s, cur_d)
        if b + 1 < NBLK:
            _stage_wait(nxt_s, nxt_d, nxt_sem)
    plsc.subcore_barrier()

    pltpu.sync_copy(acc.at[pl.ds(row0, RPT)], out_hbm.at[c, s])


@jax.jit
def _sc_segsum(x, src_r, dst_r):
    kern = pl.kernel(
        _sc_segsum_body,
        out_type=jax.ShapeDtypeStruct((NC, NS, RPT, D), jnp.float32),
        mesh=_mesh(),
        scratch_types=[
            pltpu.VMEM((BLK, K), jnp.int32),
            pltpu.VMEM((BLK, K), jnp.int32),
            pltpu.VMEM((BLK, K), jnp.int32),
            pltpu.VMEM((BLK, K), jnp.int32),
            pltpu.VMEM((K, D), jnp.float32),
            pltpu.VMEM((K, D), jnp.float32),
            pltpu.VMEM((ZR, D), jnp.float32),
            pltpu.VMEM_SHARED((N, D), jnp.float32),
            pltpu.SemaphoreType.DMA,
            pltpu.SemaphoreType.DMA,
            pltpu.SemaphoreType.DMA,
            pltpu.SemaphoreType.DMA,
        ],
    )
    dummy = jnp.zeros((K, D), jnp.float32)
    return kern(x, src_r, dst_r, dummy).reshape(NC, N, D)


BM = 2000
GRID = N // BM


def _tc_a_body(s_ref, c1_ref, cg_ref, x_ref, w_ref, b_ref, o_ref):
    ssum = s_ref[0] + s_ref[1]
    c1 = jnp.maximum(jnp.sum(c1_ref[...], axis=1), 1.0)
    m = ssum / c1[:, None]
    h = jnp.dot(m, w_ref[...], preferred_element_type=jnp.float32)
    e1 = jnp.maximum(h + b_ref[...], 0.0) + x_ref[...]
    cg = jnp.sum(cg_ref[...], axis=1) + 1.0
    dinv = lax.rsqrt(cg)
    o_ref[...] = e1 * dinv[:, None]


def _tc_b_body(s_ref, cg_ref, xs_ref, w_ref, b_ref, o_ref):
    ssum = s_ref[0] + s_ref[1]
    cg = jnp.sum(cg_ref[...], axis=1) + 1.0
    dinv = lax.rsqrt(cg)
    g = (ssum + xs_ref[...]) * dinv[:, None]
    o_ref[...] = jnp.dot(g, w_ref[...], preferred_element_type=jnp.float32) \
        + b_ref[...]


def _tc_c_body(s_ref, c3_ref, x_ref, w_ref, b_ref, o_ref):
    ssum = s_ref[0] + s_ref[1]
    c3 = jnp.maximum(jnp.sum(c3_ref[...], axis=1), 1.0)
    m = ssum / c3[:, None]
    h = jnp.dot(m, w_ref[...], preferred_element_type=jnp.float32)
    o_ref[...] = jnp.maximum(h + b_ref[...], 0.0) + x_ref[...]


def _row_spec(bm, d):
    return pl.BlockSpec((bm, d), lambda i: (i, 0))


_S_SPEC = pl.BlockSpec((NC, BM, D), lambda i: (0, i, 0))
_CNT_SPEC = pl.BlockSpec((BM, NW), lambda i: (i, 0))
_W_SPEC = pl.BlockSpec((D, D), lambda i: (0, 0))
_B_SPEC = pl.BlockSpec((1, D), lambda i: (0, 0))
_X_SPEC = _row_spec(BM, D)
_OUT = jax.ShapeDtypeStruct((N, D), jnp.float32)


@jax.jit
def _tc_stage_a(s_parts, c1_parts, cg_parts, x, w, b):
    return pl.pallas_call(
        _tc_a_body,
        grid=(GRID,),
        compiler_params=pltpu.CompilerParams(
            dimension_semantics=("parallel",)),
        in_specs=[_S_SPEC, _CNT_SPEC, _CNT_SPEC, _X_SPEC, _W_SPEC, _B_SPEC],
        out_specs=_X_SPEC,
        out_shape=_OUT,
    )(s_parts, c1_parts, cg_parts, x, w, b.reshape(1, D))


@jax.jit
def _tc_stage_b(s_parts, cg_parts, xs, w, b):
    return pl.pallas_call(
        _tc_b_body,
        grid=(GRID,),
        compiler_params=pltpu.CompilerParams(
            dimension_semantics=("parallel",)),
        in_specs=[_S_SPEC, _CNT_SPEC, _X_SPEC, _W_SPEC, _B_SPEC],
        out_specs=_X_SPEC,
        out_shape=_OUT,
    )(s_parts, cg_parts, xs, w, b.reshape(1, D))


@jax.jit
def _tc_stage_c(s_parts, c3_parts, x, w, b):
    return pl.pallas_call(
        _tc_c_body,
        grid=(GRID,),
        compiler_params=pltpu.CompilerParams(
            dimension_semantics=("parallel",)),
        in_specs=[_S_SPEC, _CNT_SPEC, _X_SPEC, _W_SPEC, _B_SPEC],
        out_specs=_X_SPEC,
        out_shape=_OUT,
    )(s_parts, c3_parts, x, w, b.reshape(1, D))


def kernel(embedding, down2up_path, same_level_edge_index, up2down_edge_index,
           W_d2u, b_d2u, W_gcn, b_gcn, W_u2d, b_u2d):
    e_d2u = down2up_path.astype(jnp.int32)
    e_gcn = same_level_edge_index.astype(jnp.int32)
    e_u2d = up2down_edge_index.astype(jnp.int32)

    def _split(e):
        return (e[0].reshape(NW * NBLK, BLK, K), e[1].reshape(NW * NBLK, BLK, K))

    src1, dst1 = _split(e_d2u)
    srcg, dstg = _split(e_gcn)
    src3, dst3 = _split(e_u2d)

    cnts = _sc_counts(
        e_d2u[1].reshape(NW, EPW // 16, 16),
        e_gcn[1].reshape(NW, EPW // 16, 16),
        e_u2d[1].reshape(NW, EPW // 16, 16),
    ).transpose(0, 2, 1)

    s1 = _sc_segsum(embedding, src1, dst1)
    xs = _tc_stage_a(s1, cnts[0], cnts[1], embedding, W_d2u, b_d2u)
    s2 = _sc_segsum(xs, srcg, dstg)
    e2 = _tc_stage_b(s2, cnts[1], xs, W_gcn, b_gcn)
    s3 = _sc_segsum(e2, src3, dst3)
    return _tc_stage_c(s3, cnts[2], e2, W_u2d, b_u2d)
